# baseline (device time: 18740 ns/iter reference)
import jax
import jax.numpy as jnp
from jax import lax
from jax.experimental import pallas as pl
from jax.experimental.pallas import tpu as pltpu

N_DEV = 4
B, H, D, BS = 16, 16, 64, 16
NB = 128
PAGES_LOCAL = 128
NEG_INF = -1e30


def kernel(Q, K, V, bt, lens):
    kp = K.transpose(1, 2, 3, 0)
    vp = V.transpose(1, 2, 3, 0)

    def body(q_ref, k_ref, v_ref, bt_ref, lens_ref, out_ref,
             kwin, vwin, comm_ref, kv_sems, send_sems, recv_sems):
        my_i = lax.axis_index("i")

        HC = H // 4
        kv_copies = []
        for c in range(4):
            hs = slice(c * HC, (c + 1) * HC)
            ck = pltpu.make_async_copy(
                k_ref.at[:, hs, :, :], kwin.at[:, hs, :, :],
                kv_sems.at[c, 0])
            cv = pltpu.make_async_copy(
                v_ref.at[:, hs, :, :], vwin.at[:, hs, :, :],
                kv_sems.at[c, 1])
            ck.start()
            cv.start()
            kv_copies.append((ck, cv))

        bt3 = bt_ref[:, :][:, :, None]
        lens3 = lens_ref[:].reshape(B, 1, 1)
        kpos = lax.broadcasted_iota(jnp.int32, (B, NB, 1), 1)
        pid = (lax.broadcasted_iota(jnp.int32, (1, 1, PAGES_LOCAL), 2)
               + my_i * PAGES_LOCAL)
        hit = (bt3 == pid) & (kpos < lens3)
        counts = jnp.sum(jnp.where(hit, 1.0, 0.0).astype(jnp.float32),
                         axis=1)
        lc = jnp.where(counts > 0.0, jnp.log(counts), NEG_INF)
        lncnt = jnp.broadcast_to(
            lc[:, None, :], (B, BS, PAGES_LOCAL)
        ).reshape(B, BS * PAGES_LOCAL)

        barrier_sem = pltpu.get_barrier_semaphore()
        for t in range(1, N_DEV):
            pl.semaphore_signal(
                barrier_sem, inc=1,
                device_id=((my_i + t) % N_DEV,),
                device_id_type=pl.DeviceIdType.MESH,
            )
        pl.semaphore_wait(barrier_sem, N_DEV - 1)

        rdmas = []

        def send_half(half):
            hs = slice(half * (H // 2), (half + 1) * (H // 2))
            for t in range(1, N_DEV):
                rdma = pltpu.make_async_remote_copy(
                    src_ref=comm_ref.at[0, hs],
                    dst_ref=comm_ref.at[t, hs],
                    send_sem=send_sems.at[t - 1, half],
                    recv_sem=recv_sems.at[t - 1, half],
                    device_id=((my_i + t) % N_DEV,),
                    device_id_type=pl.DeviceIdType.MESH,
                )
                rdma.start()
                rdmas.append(rdma)

        KEYS = BS * PAGES_LOCAL
        for h in range(H):
            if h % HC == 0:
                ck, cv = kv_copies[h // HC]
                ck.wait()
                cv.wait()
            if h == H // 2:
                send_half(0)
            q_h = q_ref[:, 0, h, :] * (D ** -0.5)
            ktr_v = jnp.transpose(
                kwin[:, h, :, :], (1, 0, 2)).reshape(D, KEYS)
            vtr_v = jnp.transpose(
                vwin[:, h, :, :], (1, 0, 2)).reshape(D, KEYS)
            s_h = lax.dot_general(
                q_h, ktr_v, (((1,), (0,)), ((), ())),
                preferred_element_type=jnp.float32,
            ) + lncnt
            m_h = jnp.max(s_h, axis=1, keepdims=True)
            p_h = jnp.exp(s_h - m_h)
            l_h = jnp.sum(p_h, axis=1, keepdims=True)
            o_h = lax.dot_general(
                p_h, vtr_v, (((1,), (1,)), ((), ())),
                preferred_element_type=jnp.float32,
            )
            comm_ref[0, h, :, 0:D] = o_h
            comm_ref[0, h, :, D:D + 1] = m_h
            comm_ref[0, h, :, D + 1:D + 2] = l_h

        send_half(1)

        acc_o = comm_ref[0, :, :, 0:D]
        acc_m = comm_ref[0, :, :, D:D + 1]
        acc_l = comm_ref[0, :, :, D + 1:D + 2]
        for t in range(1, N_DEV):
            rdmas[t - 1].wait()
            rdmas[t + 2].wait()
            o_in = comm_ref[t, :, :, 0:D]
            m_in = comm_ref[t, :, :, D:D + 1]
            l_in = comm_ref[t, :, :, D + 1:D + 2]
            m_new = jnp.maximum(acc_m, m_in)
            a = jnp.exp(acc_m - m_new)
            bweight = jnp.exp(m_in - m_new)
            acc_o = acc_o * a + o_in * bweight
            acc_l = acc_l * a + l_in * bweight
            acc_m = m_new

        res = acc_o / acc_l
        out_ref[:, 0, :, :] = jnp.transpose(res, (1, 0, 2))

    return pl.pallas_call(
        body,
        out_shape=jax.ShapeDtypeStruct((B, 1, H, D), jnp.float32),
        in_specs=[
            pl.BlockSpec(memory_space=pltpu.VMEM),
            pl.BlockSpec(memory_space=pl.ANY),
            pl.BlockSpec(memory_space=pl.ANY),
            pl.BlockSpec(memory_space=pltpu.VMEM),
            pl.BlockSpec(memory_space=pltpu.VMEM),
        ],
        out_specs=pl.BlockSpec(memory_space=pltpu.VMEM),
        scratch_shapes=[
            pltpu.VMEM((BS, H, D, PAGES_LOCAL), jnp.float32),
            pltpu.VMEM((BS, H, D, PAGES_LOCAL), jnp.float32),
            pltpu.VMEM((N_DEV, H, B, PAGES_LOCAL), jnp.float32),
            pltpu.SemaphoreType.DMA((4, 2)),
            pltpu.SemaphoreType.DMA((N_DEV - 1, 2)),
            pltpu.SemaphoreType.DMA((N_DEV - 1, 2)),
        ],
        compiler_params=pltpu.CompilerParams(
            collective_id=0, vmem_limit_bytes=60 * 1024 * 1024),
    )(Q, kp, vp, bt, lens)


# device time: 18563 ns/iter; 1.0095x vs baseline; 1.0095x over previous
import jax
import jax.numpy as jnp
from jax import lax
from jax.experimental import pallas as pl
from jax.experimental.pallas import tpu as pltpu

N_DEV = 4
B, H, D, BS = 16, 16, 64, 16
NB = 128
PAGES_LOCAL = 128
NEG_INF = -1e30


def kernel(Q, K, V, bt, lens):
    kp = K.transpose(1, 2, 3, 0)
    vp = V.transpose(1, 2, 3, 0)

    def body(q_ref, k_ref, v_ref, bt_ref, lens_ref, out_ref,
             kwin, vwin, comm_ref, kv_sems, send_sems, recv_sems):
        my_i = lax.axis_index("i")

        HC = H // 4
        kv_copies = []
        for c in range(4):
            hs = slice(c * HC, (c + 1) * HC)
            ck = pltpu.make_async_copy(
                k_ref.at[:, hs, :, :], kwin.at[:, hs, :, :],
                kv_sems.at[c, 0])
            cv = pltpu.make_async_copy(
                v_ref.at[:, hs, :, :], vwin.at[:, hs, :, :],
                kv_sems.at[c, 1])
            ck.start()
            cv.start()
            kv_copies.append((ck, cv))

        bt3 = bt_ref[:, :][:, :, None]
        lens3 = lens_ref[:].reshape(B, 1, 1)
        kpos = lax.broadcasted_iota(jnp.int32, (B, NB, 1), 1)
        pid = (lax.broadcasted_iota(jnp.int32, (1, 1, PAGES_LOCAL), 2)
               + my_i * PAGES_LOCAL)
        hit = (bt3 == pid) & (kpos < lens3)
        counts = jnp.sum(jnp.where(hit, 1.0, 0.0).astype(jnp.float32),
                         axis=1)
        lc = jnp.where(counts > 0.0, jnp.log(counts), NEG_INF)
        lncnt = jnp.broadcast_to(
            lc[:, None, :], (B, BS, PAGES_LOCAL)
        ).reshape(B, BS * PAGES_LOCAL)

        barrier_sem = pltpu.get_barrier_semaphore()
        for t in range(1, N_DEV):
            pl.semaphore_signal(
                barrier_sem, inc=1,
                device_id=((my_i + t) % N_DEV,),
                device_id_type=pl.DeviceIdType.MESH,
            )
        pl.semaphore_wait(barrier_sem, N_DEV - 1)

        rdmas = []

        def send_quarter(qtr):
            hs = slice(qtr * (H // 4), (qtr + 1) * (H // 4))
            for t in range(1, N_DEV):
                rdma = pltpu.make_async_remote_copy(
                    src_ref=comm_ref.at[0, hs],
                    dst_ref=comm_ref.at[t, hs],
                    send_sem=send_sems.at[t - 1, qtr],
                    recv_sem=recv_sems.at[t - 1, qtr],
                    device_id=((my_i + t) % N_DEV,),
                    device_id_type=pl.DeviceIdType.MESH,
                )
                rdma.start()
                rdmas.append(rdma)

        KEYS = BS * PAGES_LOCAL
        for h in range(H):
            if h % HC == 0:
                ck, cv = kv_copies[h // HC]
                ck.wait()
                cv.wait()
            if h > 0 and h % (H // 4) == 0:
                send_quarter(h // (H // 4) - 1)
            q_h = q_ref[:, 0, h, :] * (D ** -0.5)
            ktr_v = jnp.transpose(
                kwin[:, h, :, :], (1, 0, 2)).reshape(D, KEYS)
            vtr_v = jnp.transpose(
                vwin[:, h, :, :], (1, 0, 2)).reshape(D, KEYS)
            s_h = lax.dot_general(
                q_h, ktr_v, (((1,), (0,)), ((), ())),
                preferred_element_type=jnp.float32,
            ) + lncnt
            m_h = jnp.max(s_h, axis=1, keepdims=True)
            p_h = jnp.exp(s_h - m_h)
            l_h = jnp.sum(p_h, axis=1, keepdims=True)
            o_h = lax.dot_general(
                p_h, vtr_v, (((1,), (1,)), ((), ())),
                preferred_element_type=jnp.float32,
            )
            comm_ref[0, h, :, 0:D] = o_h
            comm_ref[0, h, :, D:D + 1] = m_h
            comm_ref[0, h, :, D + 1:D + 2] = l_h

        send_quarter(3)

        acc_o = comm_ref[0, :, :, 0:D]
        acc_m = comm_ref[0, :, :, D:D + 1]
        acc_l = comm_ref[0, :, :, D + 1:D + 2]
        for t in range(1, N_DEV):
            for qtr in range(4):
                rdmas[qtr * (N_DEV - 1) + t - 1].wait()
            o_in = comm_ref[t, :, :, 0:D]
            m_in = comm_ref[t, :, :, D:D + 1]
            l_in = comm_ref[t, :, :, D + 1:D + 2]
            m_new = jnp.maximum(acc_m, m_in)
            a = jnp.exp(acc_m - m_new)
            bweight = jnp.exp(m_in - m_new)
            acc_o = acc_o * a + o_in * bweight
            acc_l = acc_l * a + l_in * bweight
            acc_m = m_new

        res = acc_o / acc_l
        out_ref[:, 0, :, :] = jnp.transpose(res, (1, 0, 2))

    return pl.pallas_call(
        body,
        out_shape=jax.ShapeDtypeStruct((B, 1, H, D), jnp.float32),
        in_specs=[
            pl.BlockSpec(memory_space=pltpu.VMEM),
            pl.BlockSpec(memory_space=pl.ANY),
            pl.BlockSpec(memory_space=pl.ANY),
            pl.BlockSpec(memory_space=pltpu.VMEM),
            pl.BlockSpec(memory_space=pltpu.VMEM),
        ],
        out_specs=pl.BlockSpec(memory_space=pltpu.VMEM),
        scratch_shapes=[
            pltpu.VMEM((BS, H, D, PAGES_LOCAL), jnp.float32),
            pltpu.VMEM((BS, H, D, PAGES_LOCAL), jnp.float32),
            pltpu.VMEM((N_DEV, H, B, PAGES_LOCAL), jnp.float32),
            pltpu.SemaphoreType.DMA((4, 2)),
            pltpu.SemaphoreType.DMA((N_DEV - 1, 4)),
            pltpu.SemaphoreType.DMA((N_DEV - 1, 4)),
        ],
        compiler_params=pltpu.CompilerParams(
            collective_id=0, vmem_limit_bytes=60 * 1024 * 1024),
    )(Q, kp, vp, bt, lens)
